# baseline (speedup 1.0000x reference)
import functools

import jax
import jax.numpy as jnp
from jax import lax
from jax.experimental import pallas as pl
from jax.experimental.pallas import tpu as pltpu
from jax.experimental.pallas import tpu_sc as plsc

_NC = 2
_NS = 16
_NW = _NC * _NS
_LANES = 16
_WIN = 128
_CH = 1024


def kernel(x, perm):
    B, T, N, C = x.shape
    rows = B * T * N
    half = (T * N) // _NW
    cpi = half // _CH
    x2 = x.reshape(rows, C)
    perm_flat = jnp.asarray(perm, jnp.int32).reshape(T * N)

    mesh = plsc.VectorSubcoreMesh(core_axis_name="c", subcore_axis_name="s")

    @functools.partial(
        pl.kernel,
        mesh=mesh,
        out_type=jax.ShapeDtypeStruct((rows, C), x.dtype),
        compiler_params=pltpu.CompilerParams(use_tc_tiling_on_sc=False),
        scratch_types=[
            pltpu.VMEM((half,), jnp.int32),
            pltpu.VMEM((_CH,), jnp.int32),
            pltpu.VMEM((_CH,), jnp.int32),
            pltpu.VMEM((_CH, C), jnp.float32),
            pltpu.VMEM((_CH, C), jnp.float32),
            pltpu.SemaphoreType.DMA,
            pltpu.SemaphoreType.DMA,
            pltpu.SemaphoreType.DMA,
        ],
    )
    def scramble(x_hbm, perm_hbm, out_hbm,
                 pidx, gidx0, gidx1, rows0, rows1, sem_g, sem_w0, sem_w1):
        wid = lax.axis_index("s") * _NC + lax.axis_index("c")
        t = wid // 2
        h = wid % 2
        pltpu.sync_copy(perm_hbm.at[pl.ds(t * N + h * half, half)], pidx)

        @pl.loop(0, (B * cpi) // 2)
        def _(g):
            for p, gidx, rows_v, sem_w in ((0, gidx0, rows0, sem_w0),
                                           (1, gidx1, rows1, sem_w1)):
                c = g * 2 + p
                b = c // cpi
                q = c % cpi
                off = (b * T + t) * N

                @pl.loop(0, _CH, step=_LANES)
                def _(i):
                    gidx.at[pl.ds(i, _LANES)][...] = (
                        pidx.at[pl.ds(q * _CH + i, _LANES)][...] + off)

                @pl.when(g > 0)
                def _():
                    pltpu.make_async_copy(
                        rows_v, out_hbm.at[pl.ds(0, _CH)], sem_w).wait()

                copies = []
                for w in range(0, _CH, _WIN):
                    copies.append(pltpu.async_copy(
                        x_hbm.at[gidx.at[pl.ds(w, _WIN)]],
                        rows_v.at[pl.ds(w, _WIN)], sem_g))
                for cp in copies:
                    cp.wait()

                pltpu.async_copy(
                    rows_v, out_hbm.at[pl.ds(off + h * half + q * _CH, _CH)],
                    sem_w)

        pltpu.make_async_copy(rows0, out_hbm.at[pl.ds(0, _CH)], sem_w0).wait()
        pltpu.make_async_copy(rows1, out_hbm.at[pl.ds(0, _CH)], sem_w1).wait()

    out2 = scramble(x2, perm_flat)
    return out2.reshape(B, T, N, C)


# speedup vs baseline: 1.0115x; 1.0115x over previous
import functools

import jax
import jax.numpy as jnp
from jax import lax
from jax.experimental import pallas as pl
from jax.experimental.pallas import tpu as pltpu
from jax.experimental.pallas import tpu_sc as plsc

_NC = 2
_NS = 16
_NW = _NC * _NS
_WIN = 128
_CH = 1024


def kernel(x, perm):
    B, T, N, C = x.shape
    half = (T * N) // _NW
    cpi = half // _CH
    perm32 = jnp.asarray(perm, jnp.int32)

    mesh = plsc.VectorSubcoreMesh(core_axis_name="c", subcore_axis_name="s")

    @functools.partial(
        pl.kernel,
        mesh=mesh,
        out_type=jax.ShapeDtypeStruct((B, T, N, C), x.dtype),
        compiler_params=pltpu.CompilerParams(use_tc_tiling_on_sc=False),
        scratch_types=[
            pltpu.VMEM((half,), jnp.int32),
            pltpu.VMEM((_CH, C), jnp.float32),
            pltpu.VMEM((_CH, C), jnp.float32),
            pltpu.SemaphoreType.DMA,
            pltpu.SemaphoreType.DMA,
            pltpu.SemaphoreType.DMA,
        ],
    )
    def scramble(x_hbm, perm_hbm, out_hbm,
                 pidx, rows0, rows1, sem_g, sem_w0, sem_w1):
        wid = lax.axis_index("s") * _NC + lax.axis_index("c")
        t = wid // 2
        h = wid % 2
        pltpu.sync_copy(perm_hbm.at[t, pl.ds(h * half, half)], pidx)

        @pl.loop(0, (B * cpi) // 2)
        def _(g):
            for p, rows_v, sem_w in ((0, rows0, sem_w0), (1, rows1, sem_w1)):
                c = g * 2 + p
                b = c // cpi
                q = c % cpi
                v0 = h * half + q * _CH

                @pl.when(g > 0)
                def _():
                    pltpu.make_async_copy(
                        rows_v, out_hbm.at[0, 0, pl.ds(0, _CH)], sem_w).wait()

                copies = []
                for w in range(0, _CH, _WIN):
                    copies.append(pltpu.async_copy(
                        x_hbm.at[b, t].at[pidx.at[pl.ds(q * _CH + w, _WIN)]],
                        rows_v.at[pl.ds(w, _WIN)], sem_g))
                for cp in copies:
                    cp.wait()

                pltpu.async_copy(
                    rows_v, out_hbm.at[b, t].at[pl.ds(v0, _CH)], sem_w)

        pltpu.make_async_copy(
            rows0, out_hbm.at[0, 0, pl.ds(0, _CH)], sem_w0).wait()
        pltpu.make_async_copy(
            rows1, out_hbm.at[0, 0, pl.ds(0, _CH)], sem_w1).wait()

    return scramble(x, perm32)


# speedup vs baseline: 2.0734x; 2.0498x over previous
import functools

import jax
import jax.numpy as jnp
from jax import lax
from jax.experimental import pallas as pl
from jax.experimental import pallas as pl
from jax.experimental.pallas import tpu as pltpu
from jax.experimental.pallas import tpu_sc as plsc

_NC = 2
_NS = 16
_NW = _NC * _NS
_L = 16


def kernel(x, perm):
    B, T, N, C = x.shape
    nb = N // 128
    cblk = C // 8
    units = (B * T * cblk) // _NW

    xT = jnp.swapaxes(x, 2, 3)
    perm3 = jnp.asarray(perm, jnp.int32).reshape(T, nb, 128)

    mesh = plsc.VectorSubcoreMesh(core_axis_name="c", subcore_axis_name="s")

    @functools.partial(
        pl.kernel,
        mesh=mesh,
        out_type=jax.ShapeDtypeStruct((B, T, C, N), x.dtype),
        compiler_params=pltpu.CompilerParams(needs_layout_passes=False),
        scratch_types=[
            pltpu.VMEM((nb, 128), jnp.int32),
            pltpu.VMEM((nb, 8, 128), jnp.float32),
            pltpu.VMEM((nb, 8, 128), jnp.float32),
            pltpu.SemaphoreType.DMA,
            pltpu.SemaphoreType.DMA,
        ],
    )
    def scramble(xT_hbm, perm_hbm, outT_hbm, pbuf, sbuf, obuf, sem_i, sem_o):
        wid = lax.axis_index("s") * _NC + lax.axis_index("c")

        @pl.loop(0, units)
        def _(ui):
            unit = ui * _NW + wid
            bt = unit // cblk
            cb = unit % cblk
            b = bt // T
            t = bt % T

            pltpu.sync_copy(perm_hbm.at[t], pbuf)
            in_copies = []
            for j in range(nb):
                in_copies.append(pltpu.async_copy(
                    xT_hbm.at[b, t].at[pl.ds(cb * 8, 8), pl.ds(j * 128, 128)],
                    sbuf.at[j], sem_i))
            for cp in in_copies:
                cp.wait()

            @pl.loop(0, nb)
            def _(vb):
                for k in range(128 // _L):
                    pv = pbuf[vb, pl.ds(k * _L, _L)]
                    i0 = pv >> 7
                    i2 = pv & 127
                    for sl in range(8):
                        i1 = jnp.full((_L,), sl, jnp.int32)
                        obuf[vb, sl, pl.ds(k * _L, _L)] = plsc.load_gather(
                            sbuf, [i0, i1, i2])

            out_copies = []
            for j in range(nb):
                out_copies.append(pltpu.async_copy(
                    obuf.at[j],
                    outT_hbm.at[b, t].at[pl.ds(cb * 8, 8), pl.ds(j * 128, 128)],
                    sem_o))
            for cp in out_copies:
                cp.wait()

    outT = scramble(xT, perm3)
    return jnp.swapaxes(outT, 2, 3)


# speedup vs baseline: 2.8702x; 1.3843x over previous
import functools

import jax
import jax.numpy as jnp
from jax import lax
from jax.experimental import pallas as pl
from jax.experimental.pallas import tpu as pltpu
from jax.experimental.pallas import tpu_sc as plsc

_NC = 2
_NS = 16
_NW = _NC * _NS
_L = 16


def kernel(x, perm):
    B, T, N, C = x.shape
    nb = N // 128
    cblk = C // 8
    units = (B * T * cblk) // _NW
    hb = nb // 2

    V = x.reshape(B, T, nb, 128, cblk, 8).transpose(0, 1, 4, 2, 5, 3)
    perm3 = jnp.asarray(perm, jnp.int32).reshape(T, nb, 128)

    mesh = plsc.VectorSubcoreMesh(core_axis_name="c", subcore_axis_name="s")

    @functools.partial(
        pl.kernel,
        mesh=mesh,
        out_type=jax.ShapeDtypeStruct((B, T, cblk, nb, 8, 128), x.dtype),
        compiler_params=pltpu.CompilerParams(needs_layout_passes=False),
        scratch_types=[
            pltpu.VMEM((nb, 128), jnp.int32),
            pltpu.VMEM((nb, 128), jnp.int32),
            pltpu.VMEM((nb, 8, 128), jnp.float32),
            pltpu.VMEM((nb, 8, 128), jnp.float32),
            pltpu.VMEM((hb, 8, 128), jnp.float32),
            pltpu.VMEM((hb, 8, 128), jnp.float32),
            pltpu.SemaphoreType.DMA,
            pltpu.SemaphoreType.DMA,
            pltpu.SemaphoreType.DMA,
            pltpu.SemaphoreType.DMA,
        ],
    )
    def scramble(v_hbm, perm_hbm, o_hbm,
                 pbuf0, pbuf1, sbuf0, sbuf1, obufa, obufb,
                 sem_i, sem_p, sem_oa, sem_ob):
        wid = lax.axis_index("s") * _NC + lax.axis_index("c")

        def coords(u):
            return u // (T * cblk), (u // cblk) % T, u % cblk

        b0, t0, c0 = coords(wid)
        pltpu.async_copy(v_hbm.at[b0, t0, c0], sbuf0, sem_i)
        pltpu.async_copy(perm_hbm.at[t0], pbuf0, sem_p)

        @pl.loop(0, units // 2)
        def _(g):
            for e, pbuf, sbuf, nx_pbuf, nx_sbuf in (
                    (0, pbuf0, sbuf0, pbuf1, sbuf1),
                    (1, pbuf1, sbuf1, pbuf0, sbuf0)):
                ui = g * 2 + e
                unit = ui * _NW + wid
                b, t, cb = coords(unit)

                pltpu.make_async_copy(v_hbm.at[0, 0, 0], sbuf, sem_i).wait()
                pltpu.make_async_copy(perm_hbm.at[0], pbuf, sem_p).wait()

                @pl.when(ui < units - 1)
                def _():
                    bn, tn, cn = coords(unit + _NW)
                    pltpu.async_copy(v_hbm.at[bn, tn, cn], nx_sbuf, sem_i)
                    pltpu.async_copy(perm_hbm.at[tn], nx_pbuf, sem_p)

                for half, obuf, sem_o in ((0, obufa, sem_oa),
                                          (1, obufb, sem_ob)):
                    @pl.when(ui > 0)
                    def _():
                        pltpu.make_async_copy(
                            o_hbm.at[0, 0, 0].at[pl.ds(0, hb)], obuf,
                            sem_o).wait()

                    @pl.loop(0, hb)
                    def _(vj):
                        vb = half * hb + vj
                        for k in range(128 // _L):
                            pv = pbuf[vb, pl.ds(k * _L, _L)]
                            lo = pv & 127
                            hi = (pv >> 7) * 8
                            for sl in range(8):
                                obuf[vj, sl, pl.ds(k * _L, _L)] = (
                                    plsc.load_gather(
                                        sbuf.reshape((nb * 8, 128)),
                                        [hi + sl, lo]))

                    pltpu.async_copy(
                        obuf, o_hbm.at[b, t, cb].at[pl.ds(half * hb, hb)],
                        sem_o)

        pltpu.make_async_copy(
            o_hbm.at[0, 0, 0].at[pl.ds(0, hb)], obufa, sem_oa).wait()
        pltpu.make_async_copy(
            o_hbm.at[0, 0, 0].at[pl.ds(0, hb)], obufb, sem_ob).wait()

    out6 = scramble(V, perm3)
    return out6.transpose(0, 1, 3, 5, 2, 4).reshape(B, T, N, C)


# speedup vs baseline: 6.0371x; 2.1034x over previous
import functools

import jax
import jax.numpy as jnp
from jax import lax
from jax.experimental import pallas as pl
from jax.experimental.pallas import tpu as pltpu
from jax.experimental.pallas import tpu_sc as plsc

_NC = 2
_NS = 16
_NW = _NC * _NS
_L = 16


def kernel(x, perm):
    B, T, N, C = x.shape
    nb = N // 128
    cblk = C // 8
    units = (B * T * cblk) // _NW
    hb = nb // 2

    V = x.reshape(B, T, nb, 128, cblk, 8).transpose(0, 1, 4, 2, 5, 3)
    perm3 = jnp.asarray(perm, jnp.int32).reshape(T, nb, 128)

    mesh = plsc.VectorSubcoreMesh(core_axis_name="c", subcore_axis_name="s")

    @functools.partial(
        pl.kernel,
        mesh=mesh,
        out_type=jax.ShapeDtypeStruct((B, T, cblk, nb, 8, 128), x.dtype),
        compiler_params=pltpu.CompilerParams(needs_layout_passes=False),
        scratch_types=[
            pltpu.VMEM((nb, 128), jnp.int32),
            pltpu.VMEM((nb, 128), jnp.int32),
            pltpu.VMEM((nb, 8, 128), jnp.float32),
            pltpu.VMEM((nb, 8, 128), jnp.float32),
            pltpu.VMEM((hb, 8, 128), jnp.float32),
            pltpu.VMEM((hb, 8, 128), jnp.float32),
            pltpu.SemaphoreType.DMA,
            pltpu.SemaphoreType.DMA,
            pltpu.SemaphoreType.DMA,
            pltpu.SemaphoreType.DMA,
        ],
    )
    def scramble(v_hbm, perm_hbm, o_hbm,
                 pbuf0, pbuf1, sbuf0, sbuf1, obufa, obufb,
                 sem_i, sem_p, sem_oa, sem_ob):
        wid = lax.axis_index("s") * _NC + lax.axis_index("c")

        def coords(u):
            return u // (T * cblk), (u // cblk) % T, u % cblk

        b0, t0, c0 = coords(wid)
        pltpu.async_copy(v_hbm.at[b0, t0, c0], sbuf0, sem_i)
        pltpu.async_copy(perm_hbm.at[t0], pbuf0, sem_p)

        @pl.loop(0, units // 2)
        def _(g):
            for e, pbuf, sbuf, nx_pbuf, nx_sbuf in (
                    (0, pbuf0, sbuf0, pbuf1, sbuf1),
                    (1, pbuf1, sbuf1, pbuf0, sbuf0)):
                ui = g * 2 + e
                unit = ui * _NW + wid
                b, t, cb = coords(unit)

                pltpu.make_async_copy(v_hbm.at[0, 0, 0], sbuf, sem_i).wait()
                pltpu.make_async_copy(perm_hbm.at[0], pbuf, sem_p).wait()

                @pl.when(ui < units - 1)
                def _():
                    bn, tn, cn = coords(unit + _NW)
                    pltpu.async_copy(v_hbm.at[bn, tn, cn], nx_sbuf, sem_i)
                    pltpu.async_copy(perm_hbm.at[tn], nx_pbuf, sem_p)

                for half, obuf, sem_o in ((0, obufa, sem_oa),
                                          (1, obufb, sem_ob)):
                    @pl.when(ui > 0)
                    def _():
                        pltpu.make_async_copy(
                            o_hbm.at[0, 0, 0].at[pl.ds(0, hb)], obuf,
                            sem_o).wait()

                    @plsc.parallel_loop(0, hb, unroll=2)
                    def _(vj):
                        vb = half * hb + vj
                        for k in range(128 // _L):
                            pv = pbuf[vb, pl.ds(k * _L, _L)]
                            lo = pv & 127
                            hi = (pv >> 7) * 8
                            for sl in range(8):
                                obuf[vj, sl, pl.ds(k * _L, _L)] = (
                                    plsc.load_gather(
                                        sbuf.reshape((nb * 8, 128)),
                                        [hi + sl, lo]))

                    pltpu.async_copy(
                        obuf, o_hbm.at[b, t, cb].at[pl.ds(half * hb, hb)],
                        sem_o)

        pltpu.make_async_copy(
            o_hbm.at[0, 0, 0].at[pl.ds(0, hb)], obufa, sem_oa).wait()
        pltpu.make_async_copy(
            o_hbm.at[0, 0, 0].at[pl.ds(0, hb)], obufb, sem_ob).wait()

    out6 = scramble(V, perm3)
    return out6.transpose(0, 1, 3, 5, 2, 4).reshape(B, T, N, C)


# speedup vs baseline: 6.1728x; 1.0225x over previous
import functools

import jax
import jax.numpy as jnp
from jax import lax
from jax.experimental import pallas as pl
from jax.experimental.pallas import tpu as pltpu
from jax.experimental.pallas import tpu_sc as plsc

_NC = 2
_NS = 16
_NW = _NC * _NS
_L = 16


def kernel(x, perm):
    B, T, N, C = x.shape
    nb = N // 128
    cblk = C // 8
    units = (B * T * cblk) // _NW
    hb = nb // 2

    V = x.reshape(B, T, nb, 128, cblk, 8).transpose(0, 1, 4, 2, 5, 3)
    perm3 = jnp.asarray(perm, jnp.int32).reshape(T, nb, 128)

    mesh = plsc.VectorSubcoreMesh(core_axis_name="c", subcore_axis_name="s")

    @functools.partial(
        pl.kernel,
        mesh=mesh,
        out_type=jax.ShapeDtypeStruct((B, T, cblk, nb, 8, 128), x.dtype),
        compiler_params=pltpu.CompilerParams(needs_layout_passes=False),
        scratch_types=[
            pltpu.VMEM((nb, 128), jnp.int32),
            pltpu.VMEM((nb, 128), jnp.int32),
            pltpu.VMEM((nb, 8, 128), jnp.float32),
            pltpu.VMEM((nb, 8, 128), jnp.float32),
            pltpu.VMEM((hb, 8, 128), jnp.float32),
            pltpu.VMEM((hb, 8, 128), jnp.float32),
            pltpu.SemaphoreType.DMA,
            pltpu.SemaphoreType.DMA,
            pltpu.SemaphoreType.DMA,
            pltpu.SemaphoreType.DMA,
        ],
    )
    def scramble(v_hbm, perm_hbm, o_hbm,
                 pbuf0, pbuf1, sbuf0, sbuf1, obufa, obufb,
                 sem_i, sem_p, sem_oa, sem_ob):
        wid = lax.axis_index("s") * _NC + lax.axis_index("c")

        def coords(u):
            return u // (T * cblk), (u // cblk) % T, u % cblk

        b0, t0, c0 = coords(wid)
        pltpu.async_copy(v_hbm.at[b0, t0, c0], sbuf0, sem_i)
        pltpu.async_copy(perm_hbm.at[t0], pbuf0, sem_p)

        @pl.loop(0, units // 2)
        def _(g):
            for e, pbuf, sbuf, nx_pbuf, nx_sbuf in (
                    (0, pbuf0, sbuf0, pbuf1, sbuf1),
                    (1, pbuf1, sbuf1, pbuf0, sbuf0)):
                ui = g * 2 + e
                unit = ui * _NW + wid
                b, t, cb = coords(unit)

                pltpu.make_async_copy(v_hbm.at[0, 0, 0], sbuf, sem_i).wait()
                pltpu.make_async_copy(perm_hbm.at[0], pbuf, sem_p).wait()

                @pl.when(ui < units - 1)
                def _():
                    bn, tn, cn = coords(unit + _NW)
                    pltpu.async_copy(v_hbm.at[bn, tn, cn], nx_sbuf, sem_i)
                    pltpu.async_copy(perm_hbm.at[tn], nx_pbuf, sem_p)

                for half, obuf, sem_o in ((0, obufa, sem_oa),
                                          (1, obufb, sem_ob)):
                    @pl.when(ui > 0)
                    def _():
                        pltpu.make_async_copy(
                            o_hbm.at[0, 0, 0].at[pl.ds(0, hb)], obuf,
                            sem_o).wait()

                    @plsc.parallel_loop(0, hb, unroll=4)
                    def _(vj):
                        vb = half * hb + vj
                        for k in range(128 // _L):
                            pv = pbuf[vb, pl.ds(k * _L, _L)]
                            lo = pv & 127
                            hi = (pv >> 7) * 8
                            for sl in range(8):
                                obuf[vj, sl, pl.ds(k * _L, _L)] = (
                                    plsc.load_gather(
                                        sbuf.reshape((nb * 8, 128)),
                                        [hi + sl, lo]))

                    pltpu.async_copy(
                        obuf, o_hbm.at[b, t, cb].at[pl.ds(half * hb, hb)],
                        sem_o)

        pltpu.make_async_copy(
            o_hbm.at[0, 0, 0].at[pl.ds(0, hb)], obufa, sem_oa).wait()
        pltpu.make_async_copy(
            o_hbm.at[0, 0, 0].at[pl.ds(0, hb)], obufb, sem_ob).wait()

    out6 = scramble(V, perm3)
    return out6.transpose(0, 1, 3, 5, 2, 4).reshape(B, T, N, C)
